# baseline (speedup 1.0000x reference)
import functools

import jax
import jax.numpy as jnp
from jax import lax
from jax.experimental import pallas as pl
from jax.experimental.pallas import tpu as pltpu
from jax.experimental.pallas import tpu_sc as plsc

E = 8
D = 768
T = 2048
TM = 256
NT = 16
NPAD = NT * TM

NC = 2
NS = 16
NW = NC * NS
CHUNK = T // NW


def _prefix_sum0(a):
    n = a.shape[0]
    k = 1
    while k < n:
        pad = jnp.zeros((k, a.shape[1]), a.dtype)
        a = a + jnp.concatenate([pad, a[:-k]], axis=0)
        k *= 2
    return a


def _prefix_sum1(a):
    n = a.shape[1]
    k = 1
    while k < n:
        pad = jnp.zeros((a.shape[0], k), a.dtype)
        a = a + jnp.concatenate([pad, a[:, :-k]], axis=1)
        k *= 2
    return a


def _router_body(x_ref, gw_ref, gb_ref,
                 dst_ref, te_ref, act_ref, avgp_ref, aux_ref):
    x = x_ref[...]
    gw = gw_ref[...]
    gb = gb_ref[...]
    logits = jnp.dot(x, gw, preferred_element_type=jnp.float32) + gb
    m = jnp.max(logits, axis=1, keepdims=True)
    ex = jnp.exp(logits - m)
    probs = ex / jnp.sum(ex, axis=1, keepdims=True)

    cols = lax.broadcasted_iota(jnp.int32, (T, E), 1)
    pmax = jnp.max(probs, axis=1, keepdims=True)
    idx = jnp.min(jnp.where(probs >= pmax, cols, E), axis=1, keepdims=True)
    one_hot = (cols == idx).astype(jnp.float32)

    counts = jnp.sum(one_hot, axis=0, keepdims=True)
    ranks_incl = _prefix_sum0(one_hot)
    rank = jnp.sum((ranks_incl - 1.0) * one_hot, axis=1, keepdims=True)

    pc = jnp.ceil(counts / TM) * TM
    ends = _prefix_sum1(pc)
    offs = ends - pc
    off_t = jnp.sum(offs * one_hot, axis=1, keepdims=True)
    dst_ref[...] = (off_t + rank).astype(jnp.int32).reshape(T)

    psel = jnp.sum(probs * one_hot, axis=0, keepdims=True)
    avgp = jnp.where(counts > 0, psel / jnp.maximum(counts, 1.0), 0.0)
    avgp_ref[...] = jnp.broadcast_to(avgp.reshape(E, 1), (E, 128))

    tile_start = (lax.broadcasted_iota(jnp.int32, (NT, E), 0)
                  .astype(jnp.float32) * TM)
    te = jnp.sum((tile_start >= jnp.broadcast_to(ends, (NT, E)))
                 .astype(jnp.int32), axis=1, keepdims=True)
    total = ends[0:1, E - 1:E]
    te_ref[...] = jnp.minimum(te, E - 1)
    act_ref[...] = (total / TM).astype(jnp.int32)

    usage = counts / float(T)
    aux_ref[...] = jnp.sum(usage * usage, keepdims=True).reshape(1, 1) * E


def _router(xf, gate_W, gate_b):
    return pl.pallas_call(
        _router_body,
        out_shape=(
            jax.ShapeDtypeStruct((T,), jnp.int32),
            jax.ShapeDtypeStruct((NT, 1), jnp.int32),
            jax.ShapeDtypeStruct((1, 1), jnp.int32),
            jax.ShapeDtypeStruct((E, 128), jnp.float32),
            jax.ShapeDtypeStruct((1, 1), jnp.float32),
        ),
    )(xf, gate_W, gate_b.reshape(1, E))


@functools.cache
def _sc_kernels():
    mesh = plsc.VectorSubcoreMesh(core_axis_name="c", subcore_axis_name="s")
    scratch = [
        pltpu.VMEM((CHUNK,), jnp.int32),
        pltpu.VMEM((CHUNK, D), jnp.float32),
        pltpu.SemaphoreType.DMA,
        pltpu.SemaphoreType.DMA,
    ]

    @functools.partial(
        pl.kernel,
        out_type=jax.ShapeDtypeStruct((NPAD, D), jnp.float32),
        mesh=mesh,
        scratch_types=scratch,
    )
    def sc_scatter(x_hbm, dst_hbm, xs_hbm, idx_v, rows_v, sem, sem2):
        wid = lax.axis_index("s") * NC + lax.axis_index("c")
        base = wid * CHUNK
        cp_idx = pltpu.async_copy(dst_hbm.at[pl.ds(base, CHUNK)], idx_v, sem2)
        cp_rows = pltpu.async_copy(x_hbm.at[0, pl.ds(base, CHUNK)], rows_v, sem)
        cp_idx.wait()
        cp_rows.wait()
        pltpu.async_copy(rows_v, xs_hbm.at[idx_v], sem).wait()

    @functools.partial(
        pl.kernel,
        out_type=jax.ShapeDtypeStruct((T, D), jnp.float32),
        mesh=mesh,
        scratch_types=scratch,
    )
    def sc_gather(ys_hbm, dst_hbm, out_hbm, idx_v, rows_v, sem, sem2):
        wid = lax.axis_index("s") * NC + lax.axis_index("c")
        base = wid * CHUNK
        pltpu.async_copy(dst_hbm.at[pl.ds(base, CHUNK)], idx_v, sem2).wait()
        pltpu.async_copy(ys_hbm.at[idx_v], rows_v, sem).wait()
        pltpu.sync_copy(rows_v, out_hbm.at[pl.ds(base, CHUNK)])

    return sc_scatter, sc_gather


def _clamp(i, na):
    return jnp.minimum(i, na[0] - 1)


def _mlp_body(te_ref, na_ref, xs_ref, w1_ref, b1_ref, w2_ref, b2_ref,
              ap_ref, out_ref):
    i = pl.program_id(0)

    @pl.when(i < na_ref[0])
    def _():
        xb = xs_ref[...]
        h = jnp.dot(xb, w1_ref[0], preferred_element_type=jnp.float32)
        h = jnp.maximum(h + b1_ref[0], 0.0)
        y = jnp.dot(h, w2_ref[0], preferred_element_type=jnp.float32)
        y = y + b2_ref[0]
        out_ref[...] = y * ap_ref[0, 0:1, 0:1]


def _mlp(te, act, xs, W1, b1, W2, b2, avgp):
    grid_spec = pltpu.PrefetchScalarGridSpec(
        num_scalar_prefetch=2,
        grid=(NT,),
        in_specs=[
            pl.BlockSpec((TM, D), lambda i, te, na: (_clamp(i, na), 0)),
            pl.BlockSpec((1, D, D), lambda i, te, na: (te[_clamp(i, na)], 0, 0)),
            pl.BlockSpec((1, 1, D), lambda i, te, na: (te[_clamp(i, na)], 0, 0)),
            pl.BlockSpec((1, D, D), lambda i, te, na: (te[_clamp(i, na)], 0, 0)),
            pl.BlockSpec((1, 1, D), lambda i, te, na: (te[_clamp(i, na)], 0, 0)),
            pl.BlockSpec((1, 1, 128), lambda i, te, na: (te[_clamp(i, na)], 0, 0)),
        ],
        out_specs=pl.BlockSpec((TM, D), lambda i, te, na: (_clamp(i, na), 0)),
    )
    return pl.pallas_call(
        _mlp_body,
        grid_spec=grid_spec,
        out_shape=jax.ShapeDtypeStruct((NPAD, D), jnp.float32),
    )(te, act, xs, W1, b1.reshape(E, 1, D), W2, b2.reshape(E, 1, D),
      avgp.reshape(E, 1, 128))


def kernel(x, gate_W, gate_b, W1, b1, W2, b2):
    B, Tn, Dn = x.shape
    xf = x.reshape(Tn, Dn)
    dst, te2, act2, avgp, aux = _router(xf, gate_W, gate_b)
    te = te2.reshape(NT)
    act = act2.reshape(1)
    sc_scatter, sc_gather = _sc_kernels()
    xs = sc_scatter(x, dst)
    ys = _mlp(te, act, xs, W1, b1, W2, b2, avgp)
    out = sc_gather(ys, dst)
    return out.reshape(B, Tn, Dn), aux[0, 0]


# speedup vs baseline: 1.1198x; 1.1198x over previous
import functools

import jax
import jax.numpy as jnp
from jax import lax
from jax.experimental import pallas as pl
from jax.experimental.pallas import tpu as pltpu
from jax.experimental.pallas import tpu_sc as plsc

E = 8
D = 768
T = 2048
TM = 384
NT = 14
NPAD = NT * TM

NC = 2
NS = 16
NW = NC * NS
CHUNK = T // NW


def _prefix_sum0(a):
    n = a.shape[0]
    k = 1
    while k < n:
        pad = jnp.zeros((k, a.shape[1]), a.dtype)
        a = a + jnp.concatenate([pad, a[:-k]], axis=0)
        k *= 2
    return a


def _prefix_sum1(a):
    n = a.shape[1]
    k = 1
    while k < n:
        pad = jnp.zeros((a.shape[0], k), a.dtype)
        a = a + jnp.concatenate([pad, a[:, :-k]], axis=1)
        k *= 2
    return a


def _router_body(x_ref, gw_ref, gb_ref,
                 dst_ref, te_ref, act_ref, avgp_ref, aux_ref):
    x = x_ref[0]
    gw = gw_ref[...]
    gb = gb_ref[...]
    logits = jnp.dot(x, gw, preferred_element_type=jnp.float32) + gb
    m = jnp.max(logits, axis=1, keepdims=True)
    ex = jnp.exp(logits - m)
    probs = ex / jnp.sum(ex, axis=1, keepdims=True)

    cols = lax.broadcasted_iota(jnp.int32, (T, E), 1)
    pmax = jnp.max(probs, axis=1, keepdims=True)
    idx = jnp.min(jnp.where(probs >= pmax, cols, E), axis=1, keepdims=True)
    one_hot = (cols == idx).astype(jnp.float32)

    counts = jnp.sum(one_hot, axis=0, keepdims=True)
    ranks_incl = _prefix_sum0(one_hot)
    rank = jnp.sum((ranks_incl - 1.0) * one_hot, axis=1, keepdims=True)

    pc = jnp.ceil(counts / TM) * TM
    ends = _prefix_sum1(pc)
    offs = ends - pc
    off_t = jnp.sum(offs * one_hot, axis=1, keepdims=True)
    dst_ref[...] = (off_t + rank).astype(jnp.int32).reshape(T)

    psel = jnp.sum(probs * one_hot, axis=0, keepdims=True)
    avgp = jnp.where(counts > 0, psel / jnp.maximum(counts, 1.0), 0.0)
    avgp_ref[...] = jnp.broadcast_to(avgp.reshape(E, 1), (E, 128))

    tile_start = (lax.broadcasted_iota(jnp.int32, (NT, E), 0)
                  .astype(jnp.float32) * TM)
    te = jnp.sum((tile_start >= jnp.broadcast_to(ends, (NT, E)))
                 .astype(jnp.int32), axis=1, keepdims=True)
    total = ends[0:1, E - 1:E]
    te_ref[...] = jnp.minimum(te, E - 1)
    act_ref[...] = (total / TM).astype(jnp.int32)

    usage = counts / float(T)
    aux_ref[...] = jnp.sum(usage * usage, keepdims=True).reshape(1, 1) * E


def _router(xf, gate_W, gate_b):
    return pl.pallas_call(
        _router_body,
        in_specs=[
            pl.BlockSpec((1, T, D), lambda: (0, 0, 0)),
            pl.BlockSpec((D, E), lambda: (0, 0)),
            pl.BlockSpec((1, E), lambda: (0, 0)),
        ],
        out_shape=(
            jax.ShapeDtypeStruct((T,), jnp.int32),
            jax.ShapeDtypeStruct((NT, 1), jnp.int32),
            jax.ShapeDtypeStruct((1, 1), jnp.int32),
            jax.ShapeDtypeStruct((E, 128), jnp.float32),
            jax.ShapeDtypeStruct((1, 1), jnp.float32),
        ),
    )(xf, gate_W, gate_b.reshape(1, E))


@functools.cache
def _sc_kernels():
    mesh = plsc.VectorSubcoreMesh(core_axis_name="c", subcore_axis_name="s")
    scratch = [
        pltpu.VMEM((CHUNK,), jnp.int32),
        pltpu.VMEM((CHUNK, D), jnp.float32),
        pltpu.SemaphoreType.DMA,
        pltpu.SemaphoreType.DMA,
    ]

    @functools.partial(
        pl.kernel,
        out_type=jax.ShapeDtypeStruct((NPAD, D), jnp.float32),
        mesh=mesh,
        scratch_types=scratch,
    )
    def sc_scatter(x_hbm, dst_hbm, xs_hbm, idx_v, rows_v, sem, sem2):
        wid = lax.axis_index("s") * NC + lax.axis_index("c")
        base = wid * CHUNK
        cp_idx = pltpu.async_copy(dst_hbm.at[pl.ds(base, CHUNK)], idx_v, sem2)
        cp_rows = pltpu.async_copy(x_hbm.at[0, pl.ds(base, CHUNK)], rows_v, sem)
        cp_idx.wait()
        cp_rows.wait()
        pltpu.async_copy(rows_v, xs_hbm.at[idx_v], sem).wait()

    @functools.partial(
        pl.kernel,
        out_type=jax.ShapeDtypeStruct((T, D), jnp.float32),
        mesh=mesh,
        scratch_types=scratch,
    )
    def sc_gather(ys_hbm, dst_hbm, out_hbm, idx_v, rows_v, sem, sem2):
        wid = lax.axis_index("s") * NC + lax.axis_index("c")
        base = wid * CHUNK
        pltpu.async_copy(dst_hbm.at[pl.ds(base, CHUNK)], idx_v, sem2).wait()
        pltpu.async_copy(ys_hbm.at[idx_v], rows_v, sem).wait()
        pltpu.sync_copy(rows_v, out_hbm.at[pl.ds(base, CHUNK)])

    return sc_scatter, sc_gather


def _clamp(i, na):
    return jnp.minimum(i, na[0] - 1)


def _mlp_body(te_ref, na_ref, xs_ref, w1_ref, b1_ref, w2_ref, b2_ref,
              ap_ref, out_ref):
    i = pl.program_id(0)

    @pl.when(i < na_ref[0])
    def _():
        xb = xs_ref[...]
        h = jnp.dot(xb, w1_ref[0], preferred_element_type=jnp.float32)
        h = jnp.maximum(h + b1_ref[0], 0.0)
        y = jnp.dot(h, w2_ref[0], preferred_element_type=jnp.float32)
        y = y + b2_ref[0]
        out_ref[...] = y * ap_ref[0, 0:1, 0:1]


def _mlp(te, act, xs, W1, b1, W2, b2, avgp):
    grid_spec = pltpu.PrefetchScalarGridSpec(
        num_scalar_prefetch=2,
        grid=(NT,),
        in_specs=[
            pl.BlockSpec((TM, D), lambda i, te, na: (_clamp(i, na), 0)),
            pl.BlockSpec((1, D, D), lambda i, te, na: (te[_clamp(i, na)], 0, 0)),
            pl.BlockSpec((1, 1, D), lambda i, te, na: (te[_clamp(i, na)], 0, 0)),
            pl.BlockSpec((1, D, D), lambda i, te, na: (te[_clamp(i, na)], 0, 0)),
            pl.BlockSpec((1, 1, D), lambda i, te, na: (te[_clamp(i, na)], 0, 0)),
            pl.BlockSpec((1, 1, 128), lambda i, te, na: (te[_clamp(i, na)], 0, 0)),
        ],
        out_specs=pl.BlockSpec((TM, D), lambda i, te, na: (_clamp(i, na), 0)),
    )
    return pl.pallas_call(
        _mlp_body,
        grid_spec=grid_spec,
        out_shape=jax.ShapeDtypeStruct((NPAD, D), jnp.float32),
    )(te, act, xs, W1, b1.reshape(E, 1, D), W2, b2.reshape(E, 1, D),
      avgp.reshape(E, 1, 128))


def kernel(x, gate_W, gate_b, W1, b1, W2, b2):
    B, Tn, Dn = x.shape
    dst, te2, act2, avgp, aux = _router(x, gate_W, gate_b)
    te = te2.reshape(NT)
    act = act2.reshape(1)
    sc_scatter, sc_gather = _sc_kernels()
    xs = sc_scatter(x, dst)
    ys = _mlp(te, act, xs, W1, b1, W2, b2, avgp)
    out = sc_gather(ys, dst)
    return out.reshape(B, Tn, Dn), aux[0, 0]
